# Initial kernel scaffold; baseline (speedup 1.0000x reference)
import jax
import jax.numpy as jnp
from jax.experimental import pallas as pl


def kernel(node_feats, edge_feats, edge_index, W1, b1, W2, b2):
    raise NotImplementedError("write your pallas kernel here")


# speedup vs baseline: 6.0725x; 6.0725x over previous
import functools

import jax
import jax.numpy as jnp
from jax import lax
from jax.experimental import pallas as pl
from jax.experimental.pallas import tpu as pltpu
from jax.experimental.pallas import tpu_sc as plsc

N_NODES = 10000
N_EDGES = 320000
D = 128
RRELU_SLOPE = (1.0 / 8.0 + 1.0 / 3.0) / 2.0

NUM_CORES = 2
NUM_SUBCORES = 16
TILE_ROW_BASE = 624
TILE_ROW_SPAN = 640
CHUNK_EDGES = 256
IDX_ROWS = CHUNK_EDGES // 128
N_CHUNKS = N_EDGES // CHUNK_EDGES


def _sc_body(src2d, dst2d, nf, ef, g_out, e_out, cnt_out,
             sidx, didx, rows, ones, zbuf, acc, cnt_sh, sem):
  cid = lax.axis_index("c")
  sid = lax.axis_index("s")

  zero16 = jnp.zeros((16,), jnp.float32)
  one16 = jnp.ones((16,), jnp.float32)

  def _zero_rows(i, carry):
    for k in range(8):
      rows[i, pl.ds(k * 16, 16)] = zero16
    return carry
  lax.fori_loop(0, CHUNK_EDGES, _zero_rows, 0)

  def _zero_z(i, carry):
    zbuf[pl.ds(i * 16, 16)] = zero16
    return carry
  lax.fori_loop(0, TILE_ROW_SPAN // 16, _zero_z, 0)

  for k in range(8):
    ones[pl.ds(k * 16, 16)] = one16

  abase = sid * TILE_ROW_BASE
  off = 0
  while off < TILE_ROW_SPAN:
    size = min(CHUNK_EDGES, TILE_ROW_SPAN - off)
    pltpu.sync_copy(rows.at[pl.ds(0, size)],
                    acc.at[pl.ds(abase + off, size)])
    off += size
  pltpu.sync_copy(zbuf, cnt_sh.at[pl.ds(abase, TILE_ROW_SPAN)])

  plsc.subcore_barrier()

  n_chunks = (N_CHUNKS - sid + NUM_SUBCORES - 1) // NUM_SUBCORES

  ones16 = jnp.ones((16,), jnp.float32)

  def _chunk(i, carry):
    c = sid + i * NUM_SUBCORES
    ebase = c * CHUNK_EDGES
    pltpu.sync_copy(dst2d.at[c], didx)

    @pl.when(cid == 0)
    def _gather_side():
      pltpu.sync_copy(src2d.at[c], sidx)
      descs = []
      for j in range(IDX_ROWS):
        descs.append(
            pltpu.async_copy(nf.at[sidx.at[j]],
                             rows.at[pl.ds(j * 128, 128)], sem))
      for d in descs:
        d.wait()
      for j in range(IDX_ROWS):
        pltpu.sync_copy(rows.at[pl.ds(j * 128, 128)],
                        acc.at[didx.at[j]], add=True)

    @pl.when(cid == 1)
    def _edge_side():
      pltpu.sync_copy(ef.at[pl.ds(ebase, CHUNK_EDGES)], rows)
      for j in range(IDX_ROWS):
        pltpu.sync_copy(rows.at[pl.ds(j * 128, 128)],
                        acc.at[didx.at[j]], add=True)
      for j in range(IDX_ROWS):
        pltpu.sync_copy(ones, cnt_sh.at[didx.at[j]], add=True)

    return carry

  lax.fori_loop(0, n_chunks, _chunk, 0)

  plsc.subcore_barrier()

  @pl.when(cid == 0)
  def _out_g():
    pltpu.sync_copy(acc.at[pl.ds(abase, TILE_ROW_SPAN)],
                    g_out.at[pl.ds(abase, TILE_ROW_SPAN)])

  @pl.when(cid == 1)
  def _out_e():
    pltpu.sync_copy(acc.at[pl.ds(abase, TILE_ROW_SPAN)],
                    e_out.at[pl.ds(abase, TILE_ROW_SPAN)])
    pltpu.sync_copy(cnt_sh.at[pl.ds(abase, TILE_ROW_SPAN)], zbuf)
    pltpu.sync_copy(zbuf, cnt_out.at[pl.ds(abase, TILE_ROW_SPAN)])


def _segment_sums(src2d, dst2d, node_feats, edge_feats):
  mesh = plsc.VectorSubcoreMesh(
      core_axis_name="c", subcore_axis_name="s",
      num_cores=NUM_CORES, num_subcores=NUM_SUBCORES)
  f = pl.kernel(
      _sc_body,
      out_type=[
          jax.ShapeDtypeStruct((N_NODES, D), jnp.float32),
          jax.ShapeDtypeStruct((N_NODES, D), jnp.float32),
          jax.ShapeDtypeStruct((N_NODES,), jnp.float32),
      ],
      mesh=mesh,
      scratch_types=[
          pltpu.VMEM((IDX_ROWS, 128), jnp.int32),
          pltpu.VMEM((IDX_ROWS, 128), jnp.int32),
          pltpu.VMEM((CHUNK_EDGES, D), jnp.float32),
          pltpu.VMEM((128,), jnp.float32),
          pltpu.VMEM((TILE_ROW_SPAN,), jnp.float32),
          pltpu.VMEM_SHARED((N_NODES, D), jnp.float32),
          pltpu.VMEM_SHARED((N_NODES,), jnp.float32),
          pltpu.SemaphoreType.DMA,
      ],
      compiler_params=pltpu.CompilerParams(use_tc_tiling_on_sc=False),
  )
  return f(src2d, dst2d, node_feats, edge_feats)


def _tail_body(g_ref, e_ref, cntp_ref, w1a_ref, w1b_ref, w2_ref,
               b1_ref, b2_ref, out_ref):
  cnt = cntp_ref[...][:, 0]
  ms = (jnp.dot(g_ref[...], w1a_ref[...], preferred_element_type=jnp.float32)
        + jnp.dot(e_ref[...], w1b_ref[...], preferred_element_type=jnp.float32)
        + cnt[:, None] * b1_ref[...])
  nm = ms / jnp.maximum(cnt, 1.0)[:, None]
  sm = jnp.dot(nm, w2_ref[...], preferred_element_type=jnp.float32) + b2_ref[...]
  o = nm + sm
  out_ref[...] = jnp.where(o >= 0, o, o * RRELU_SLOPE)


def _tail(g, e, cntp, w1a_t, w1b_t, w2_t, b1, b2):
  blk = 1000
  grid = (N_NODES // blk,)
  return pl.pallas_call(
      _tail_body,
      grid=grid,
      in_specs=[
          pl.BlockSpec((blk, D), lambda i: (i, 0)),
          pl.BlockSpec((blk, D), lambda i: (i, 0)),
          pl.BlockSpec((blk, 1), lambda i: (i, 0)),
          pl.BlockSpec((D, D), lambda i: (0, 0)),
          pl.BlockSpec((D, D), lambda i: (0, 0)),
          pl.BlockSpec((D, D), lambda i: (0, 0)),
          pl.BlockSpec((1, D), lambda i: (0, 0)),
          pl.BlockSpec((1, D), lambda i: (0, 0)),
      ],
      out_specs=pl.BlockSpec((blk, D), lambda i: (i, 0)),
      out_shape=jax.ShapeDtypeStruct((N_NODES, D), jnp.float32),
  )(g, e, cntp, w1a_t, w1b_t, w2_t, b1, b2)


@jax.jit
def kernel(node_feats, edge_feats, edge_index, W1, b1, W2, b2):
  src2d = edge_index[0].astype(jnp.int32).reshape(N_CHUNKS, IDX_ROWS, 128)
  dst2d = edge_index[1].astype(jnp.int32).reshape(N_CHUNKS, IDX_ROWS, 128)
  g, e, cnt = _segment_sums(src2d, dst2d, node_feats, edge_feats)
  cntp = cnt.reshape(N_NODES, 1)
  w1a_t = W1[:, :D].T
  w1b_t = W1[:, D:].T
  w2_t = W2.T
  return _tail(g, e, cntp, w1a_t, w1b_t, w2_t,
               b1.reshape(1, D), b2.reshape(1, D))
